# baseline (speedup 1.0000x reference)
import functools

import jax
import jax.numpy as jnp
from jax import lax
from jax.experimental import pallas as pl
from jax.experimental.pallas import tpu as pltpu
from jax.experimental.pallas import tpu_sc as plsc

_DIM = 128
_NC = 2
_NS = 16
_NW = _NC * _NS
_CHUNK = 128
_TC_CH = 2048


def _sc_gather_fn(n_ent, n_rel):
    e_rows_w = n_ent // _NW
    r_rows_w = n_rel // _NW
    rows_w = e_rows_w + r_rows_w
    e_chunks = e_rows_w // _CHUNK
    r_chunks = r_rows_w // _CHUNK
    n_slots = e_chunks + r_chunks
    mesh = plsc.VectorSubcoreMesh(core_axis_name="c", subcore_axis_name="s")

    @functools.partial(
        pl.kernel,
        out_type=[
            jax.ShapeDtypeStruct((n_ent, _DIM), jnp.float32),
            jax.ShapeDtypeStruct((n_rel, _DIM), jnp.float32),
        ],
        mesh=mesh,
        scratch_types=[
            pltpu.VMEM((e_rows_w,), jnp.int32),
            pltpu.VMEM((r_rows_w,), jnp.int32),
            pltpu.VMEM((rows_w, _DIM), jnp.float32),
            pltpu.SemaphoreType.DMA,
            pltpu.SemaphoreType.DMA,
        ],
    )
    def gather(ent_hbm, rel_hbm, ie_hbm, ir_hbm, oe_hbm, or_hbm,
               ie_v, ir_v, rows_v, gsem, osem):
        wid = lax.axis_index("s") * _NC + lax.axis_index("c")
        pltpu.sync_copy(ie_hbm.at[pl.ds(wid * e_rows_w, e_rows_w)], ie_v)
        pltpu.sync_copy(ir_hbm.at[pl.ds(wid * r_rows_w, r_rows_w)], ir_v)
        gathers = []
        for j in range(e_chunks):
            gathers.append(pltpu.async_copy(
                ent_hbm.at[ie_v.at[pl.ds(j * _CHUNK, _CHUNK)]],
                rows_v.at[pl.ds(j * _CHUNK, _CHUNK)], gsem))
        for j in range(r_chunks):
            gathers.append(pltpu.async_copy(
                rel_hbm.at[ir_v.at[pl.ds(j * _CHUNK, _CHUNK)]],
                rows_v.at[pl.ds((e_chunks + j) * _CHUNK, _CHUNK)], gsem))
        for g in gathers:
            g.wait()
        w0 = pltpu.async_copy(
            rows_v.at[pl.ds(0, e_rows_w)],
            oe_hbm.at[pl.ds(wid * e_rows_w, e_rows_w)], osem)
        w1 = pltpu.async_copy(
            rows_v.at[pl.ds(e_rows_w, r_rows_w)],
            or_hbm.at[pl.ds(wid * r_rows_w, r_rows_w)], osem)
        w0.wait()
        w1.wait()

    return gather


def _unit(x):
    s = jnp.sum(x * x, axis=1)
    inv = lax.rsqrt(jnp.maximum(s, 1e-24))
    return x * inv[:, None]


def _tc_loss_fn(scale):
    def _tc_loss(hp_ref, tp_ref, hn_ref, tn_ref, rp_ref, rn_ref, out_ref):
        i = pl.program_id(0)
        dp = _unit(hp_ref[...]) + _unit(rp_ref[...]) - _unit(tp_ref[...])
        dn = _unit(hn_ref[...]) + _unit(rn_ref[...]) - _unit(tn_ref[...])
        sp = jnp.maximum(jnp.sum(dp * dp, axis=1), 1e-30)
        sn = jnp.maximum(jnp.sum(dn * dn, axis=1), 1e-30)
        ep = sp * lax.rsqrt(sp)
        en = sn * lax.rsqrt(sn)
        part = jnp.sum(jnp.maximum(1.0 + ep - en, 0.0))

        @pl.when(i == 0)
        def _():
            out_ref[...] = jnp.zeros((1, 1), jnp.float32)

        out_ref[...] += part.reshape(1, 1)

        if scale is not None:
            @pl.when(i == pl.num_programs(0) - 1)
            def _():
                out_ref[...] *= scale

    return _tc_loss


def _tc_call(erows, rrows, scale):
    bs = erows.shape[0] // 4
    ch = min(_TC_CH, bs)
    nb = bs // ch
    segspec = lambda s: pl.BlockSpec(
        (ch, _DIM), lambda i, s=s: (s * nb + i, 0))
    return pl.pallas_call(
        _tc_loss_fn(scale),
        grid=(nb,),
        in_specs=[segspec(0), segspec(1), segspec(2), segspec(3),
                  segspec(0), segspec(1)],
        out_specs=pl.BlockSpec((1, 1), lambda i: (0, 0)),
        out_shape=jax.ShapeDtypeStruct((1, 1), jnp.float32),
    )(erows, erows, erows, erows, rrows, rrows)


@jax.jit
def kernel(pos_triples, neg_triples, ent_emb, rel_emb):
    b = pos_triples.shape[0]
    idx_ent = jnp.concatenate([
        pos_triples[:, 0], pos_triples[:, 2],
        neg_triples[:, 0], neg_triples[:, 2],
    ])
    idx_rel = jnp.concatenate([
        pos_triples[:, 1], neg_triples[:, 1],
    ])

    erows, rrows = _sc_gather_fn(4 * b, 2 * b)(
        ent_emb, rel_emb, idx_ent, idx_rel)
    out = _tc_call(erows, rrows, 1.0 / b)
    return out[0, 0]


# speedup vs baseline: 1.0373x; 1.0373x over previous
import functools

import jax
import jax.numpy as jnp
from jax import lax
from jax.experimental import pallas as pl
from jax.experimental.pallas import tpu as pltpu
from jax.experimental.pallas import tpu_sc as plsc

_DIM = 128
_NC = 2
_NS = 16
_NW = _NC * _NS
_CHUNK = 128
_TC_CH = 2048


def _sc_gather_fn(n_ent, n_rel):
    e_rows_w = n_ent // _NW
    r_rows_w = n_rel // _NW
    rows_w = e_rows_w + r_rows_w
    e_chunks = e_rows_w // _CHUNK
    r_chunks = r_rows_w // _CHUNK
    n_slots = e_chunks + r_chunks
    mesh = plsc.VectorSubcoreMesh(core_axis_name="c", subcore_axis_name="s")

    @functools.partial(
        pl.kernel,
        out_type=[
            jax.ShapeDtypeStruct((n_ent, _DIM), jnp.float32),
            jax.ShapeDtypeStruct((n_rel, _DIM), jnp.float32),
        ],
        mesh=mesh,
        scratch_types=[
            pltpu.VMEM((rows_w,), jnp.int32),
            pltpu.VMEM((rows_w, _DIM), jnp.float32),
            pltpu.SemaphoreType.DMA,
            pltpu.SemaphoreType.DMA,
        ],
    )
    def gather(ent_hbm, rel_hbm, idx_hbm, oe_hbm, or_hbm,
               idx_v, rows_v, gsem, osem):
        wid = lax.axis_index("s") * _NC + lax.axis_index("c")
        pltpu.sync_copy(idx_hbm.at[pl.ds(wid * rows_w, rows_w)], idx_v)
        gathers = []
        for j in range(e_chunks):
            gathers.append(pltpu.async_copy(
                ent_hbm.at[idx_v.at[pl.ds(j * _CHUNK, _CHUNK)]],
                rows_v.at[pl.ds(j * _CHUNK, _CHUNK)], gsem))
        for j in range(r_chunks):
            gathers.append(pltpu.async_copy(
                rel_hbm.at[idx_v.at[pl.ds((e_chunks + j) * _CHUNK, _CHUNK)]],
                rows_v.at[pl.ds((e_chunks + j) * _CHUNK, _CHUNK)], gsem))
        for g in gathers:
            g.wait()
        w0 = pltpu.async_copy(
            rows_v.at[pl.ds(0, e_rows_w)],
            oe_hbm.at[pl.ds(wid * e_rows_w, e_rows_w)], osem)
        w1 = pltpu.async_copy(
            rows_v.at[pl.ds(e_rows_w, r_rows_w)],
            or_hbm.at[pl.ds(wid * r_rows_w, r_rows_w)], osem)
        w0.wait()
        w1.wait()

    return gather


def _unit(x):
    s = jnp.sum(x * x, axis=1)
    inv = lax.rsqrt(jnp.maximum(s, 1e-24))
    return x * inv[:, None]


def _tc_loss_fn(scale):
    def _tc_loss(hp_ref, tp_ref, hn_ref, tn_ref, rp_ref, rn_ref, out_ref):
        i = pl.program_id(0)
        dp = _unit(hp_ref[...]) + _unit(rp_ref[...]) - _unit(tp_ref[...])
        dn = _unit(hn_ref[...]) + _unit(rn_ref[...]) - _unit(tn_ref[...])
        sp = jnp.maximum(jnp.sum(dp * dp, axis=1), 1e-30)
        sn = jnp.maximum(jnp.sum(dn * dn, axis=1), 1e-30)
        ep = sp * lax.rsqrt(sp)
        en = sn * lax.rsqrt(sn)
        part = jnp.sum(jnp.maximum(1.0 + ep - en, 0.0))

        @pl.when(i == 0)
        def _():
            out_ref[...] = jnp.zeros((1, 1), jnp.float32)

        out_ref[...] += part.reshape(1, 1)

        if scale is not None:
            @pl.when(i == pl.num_programs(0) - 1)
            def _():
                out_ref[...] *= scale

    return _tc_loss


def _tc_call(erows, rrows, scale):
    bs = erows.shape[0] // 4
    ch = min(_TC_CH, bs)
    nb = bs // ch
    segspec = lambda s: pl.BlockSpec(
        (ch, _DIM), lambda i, s=s: (s * nb + i, 0))
    return pl.pallas_call(
        _tc_loss_fn(scale),
        grid=(nb,),
        in_specs=[segspec(0), segspec(1), segspec(2), segspec(3),
                  segspec(0), segspec(1)],
        out_specs=pl.BlockSpec((1, 1), lambda i: (0, 0)),
        out_shape=jax.ShapeDtypeStruct((1, 1), jnp.float32),
    )(erows, erows, erows, erows, rrows, rrows)


@jax.jit
def kernel(pos_triples, neg_triples, ent_emb, rel_emb):
    b = pos_triples.shape[0]
    idx_ent = jnp.concatenate([
        pos_triples[:, 0], pos_triples[:, 2],
        neg_triples[:, 0], neg_triples[:, 2],
    ])
    idx_rel = jnp.concatenate([
        pos_triples[:, 1], neg_triples[:, 1],
    ])
    e_rows_w = 4 * b // _NW
    r_rows_w = 2 * b // _NW
    idx_all = jnp.concatenate([
        idx_ent.reshape(_NW, e_rows_w),
        idx_rel.reshape(_NW, r_rows_w),
    ], axis=1).reshape(-1)

    erows, rrows = _sc_gather_fn(4 * b, 2 * b)(ent_emb, rel_emb, idx_all)
    out = _tc_call(erows, rrows, 1.0 / b)
    return out[0, 0]
